# baseline (speedup 1.0000x reference)
import functools

import jax
import jax.numpy as jnp
from jax import lax
from jax.experimental import pallas as pl
from jax.experimental.pallas import tpu as pltpu
from jax.experimental.pallas import tpu_sc as plsc

N = 10000
E = 320000
F = 128
NPAD = 10240

NC = 2
NS = 16
NW = NC * NS

_mesh = functools.partial(
    plsc.VectorSubcoreMesh, core_axis_name="c", subcore_axis_name="s")
_sc_params = pltpu.CompilerParams(needs_layout_passes=False)


GSUB = 80
GFIRE = 5
GCH = GSUB * GFIRE

def _gather_rows(table, idx):
    B = idx.shape[0]
    D = table.shape[1]
    per_w = B // NW
    n_ch = per_w // GCH

    @functools.partial(
        pl.kernel, mesh=_mesh(), compiler_params=_sc_params,
        out_type=jax.ShapeDtypeStruct((B, D), jnp.float32),
        scratch_types=[
            pltpu.VMEM((GCH,), jnp.int32),
            pltpu.VMEM((GCH, D), jnp.float32),
            pltpu.SemaphoreType.DMA,
        ],
    )
    def k(table_hbm, idx_hbm, out_hbm, idx_v, rows_v, sem):
        wid = lax.axis_index("s") * NC + lax.axis_index("c")
        base = wid * per_w

        def body(i, carry):
            off = base + i * GCH
            pltpu.sync_copy(idx_hbm.at[pl.ds(off, GCH)], idx_v)
            cps = [
                pltpu.async_copy(
                    table_hbm.at[idx_v.at[pl.ds(g * GSUB, GSUB)]],
                    rows_v.at[pl.ds(g * GSUB, GSUB)], sem)
                for g in range(GFIRE)
            ]
            for cp in cps:
                cp.wait()
            pltpu.sync_copy(rows_v, out_hbm.at[pl.ds(off, GCH)])
            return carry

        lax.fori_loop(0, n_ch, body, 0)

    return k(table, idx)


BE = 256

def _edge_mlp(xsxd, edge_rbf, edge_attr, edge_vec,
              W_rbf_emb, b_rbf_emb, W_edge, b_edge, W_pre, b_pre,
              W_sm1, b_sm1, W_sm2, b_sm2, W_rbf_lin):
    nblk = E // BE

    def body(xs_r, xd_r, rbf_r, ea_r, ev_r,
             wre_r, bre_r, we_r, be_r, wp_r, bp_r,
             w1_r, b1_r, w2_r, b2_r, wrl_r,
             ms_r, mv0_r, mv1_r, mv2_r):
        rbf = rbf_r[...]
        rbf_attr = jax.nn.relu(
            jnp.dot(rbf, wre_r[...], preferred_element_type=jnp.float32)
            + bre_r[...])
        ea = (jnp.dot(ea_r[...], we_r[...], preferred_element_type=jnp.float32)
              + be_r[...])
        wp = wp_r[...]
        xd = xd_r[...]
        msg = (jnp.dot(xs_r[...], wp[0:F], preferred_element_type=jnp.float32)
               + jnp.dot(xd, wp[F:2 * F], preferred_element_type=jnp.float32)
               + jnp.dot(rbf_attr, wp[2 * F:3 * F],
                         preferred_element_type=jnp.float32)
               + jnp.dot(ea, wp[3 * F:4 * F],
                         preferred_element_type=jnp.float32)
               + bp_r[...])
        h = jnp.dot(msg, w1_r[...], preferred_element_type=jnp.float32) + b1_r[...]
        h = h * jax.nn.sigmoid(h)
        so3 = jnp.dot(h, w2_r[...], preferred_element_type=jnp.float32) + b2_r[...]
        filt = so3 * jnp.dot(rbf, wrl_r[...], preferred_element_type=jnp.float32)
        gs = filt[:, 0:F]
        ge = filt[:, F:2 * F]
        ms_r[...] = filt[:, 2 * F:3 * F]
        ev = ev_r[...]
        a = xd * gs
        mv0_r[...] = a + ge * ev[:, 0:1]
        mv1_r[...] = a + ge * ev[:, 1:2]
        mv2_r[...] = a + ge * ev[:, 2:3]

    full = lambda r, c: pl.BlockSpec((r, c), lambda i: (0, 0))
    return pl.pallas_call(
        body,
        grid=(nblk,),
        in_specs=[
            pl.BlockSpec((BE, F), lambda i: (i, 0)),
            pl.BlockSpec((BE, F), lambda i: (i + nblk, 0)),
            pl.BlockSpec((BE, 20), lambda i: (i, 0)),
            pl.BlockSpec((BE, 16), lambda i: (i, 0)),
            pl.BlockSpec((BE, 3), lambda i: (i, 0)),
            full(20, F), full(1, F), full(16, F), full(1, F),
            full(4 * F, F), full(1, F),
            full(F, F), full(1, F), full(F, 3 * F), full(1, 3 * F),
            full(20, 3 * F),
        ],
        out_specs=[pl.BlockSpec((BE, F), lambda i: (i, 0))] * 4,
        out_shape=[jax.ShapeDtypeStruct((E, F), jnp.float32)] * 4,
    )(xsxd, xsxd, edge_rbf, edge_attr, edge_vec,
      W_rbf_emb, b_rbf_emb.reshape(1, F), W_edge, b_edge.reshape(1, F),
      W_pre, b_pre.reshape(1, F), W_sm1, b_sm1.reshape(1, F),
      W_sm2, b_sm2.reshape(1, 3 * F), W_rbf_lin)


NPER = 320
SCAN = 640
CAP = 4112
HIW = CAP - SCAN - 16
GB = 128
NEG = -3.0e38

def _segment_agg(dst, ms):
    n_scan = E // SCAN

    @functools.partial(
        pl.kernel, mesh=_mesh(), compiler_params=_sc_params,
        out_type=(
            jax.ShapeDtypeStruct((NPAD, F), jnp.float32),
            jax.ShapeDtypeStruct((NPAD, F), jnp.float32),
            jax.ShapeDtypeStruct((NPAD,), jnp.float32),
        ),
        scratch_types=[
            pltpu.VMEM((NPER, F), jnp.float32),
            pltpu.VMEM((NPER, F), jnp.float32),
            pltpu.VMEM((NPER + 16,), jnp.float32),
            pltpu.VMEM((SCAN,), jnp.int32),
            pltpu.VMEM((CAP,), jnp.int32),
            pltpu.VMEM((CAP,), jnp.int32),
            pltpu.VMEM((GB, F), jnp.float32),
            pltpu.SemaphoreType.DMA,
        ],
    )
    def k(dst_hbm, ms_hbm, osum, omax, odeg,
          asum, amax, adeg, scanb, ids, rows, gbuf, sem):
        wid = lax.axis_index("s") * NC + lax.axis_index("c")
        lo = wid * NPER
        hi = lo + NPER

        zero16 = jnp.zeros((16,), jnp.float32)
        neg16 = jnp.full((16,), NEG, jnp.float32)
        zi16 = jnp.zeros((16,), jnp.int32)

        def init_acc(i, carry):
            for q in range(F // 16):
                asum[i, pl.ds(q * 16, 16)] = zero16
                amax[i, pl.ds(q * 16, 16)] = neg16
            return carry
        lax.fori_loop(0, NPER, init_acc, 0)

        def init_deg(i, carry):
            adeg[pl.ds(i * 16, 16)] = zero16
            return carry
        lax.fori_loop(0, (NPER + 16) // 16, init_deg, 0)

        def init_ids(i, carry):
            ids[pl.ds(i * 16, 16)] = zi16
            return carry
        lax.fori_loop(0, CAP // 16, init_ids, 0)

        def drain(cnt):
            nsub = (cnt + GB - 1) // GB

            def sub(p, carry):
                base_r = p * GB
                pltpu.async_copy(
                    ms_hbm.at[ids.at[pl.ds(base_r, GB)]], gbuf, sem).wait()
                nr = jnp.minimum(GB, cnt - base_r)

                def rmw(r, c2):
                    row = rows[pl.ds(base_r + r, 16)][0]
                    for q in range(F // 16):
                        val = gbuf[r, pl.ds(q * 16, 16)]
                        am = amax[row, pl.ds(q * 16, 16)]
                        amax[row, pl.ds(q * 16, 16)] = jnp.maximum(am, val)
                        sm = asum[row, pl.ds(q * 16, 16)]
                        asum[row, pl.ds(q * 16, 16)] = sm + val
                    adeg[pl.ds(row, 16)] = adeg[pl.ds(row, 16)] + e0
                    return c2
                lax.fori_loop(0, nr, rmw, 0)
                return carry
            lax.fori_loop(0, nsub, sub, 0)
            return 0

        iota16 = lax.iota(jnp.int32, 16)
        e0 = (iota16 == 0).astype(jnp.float32)

        def scan_chunk(i, off):
            pltpu.sync_copy(dst_hbm.at[pl.ds(i * SCAN, SCAN)], scanb)

            def vreg4(j4, off2):
                for u in range(4):
                    j = j4 * 4 + u
                    d = scanb[pl.ds(j * 16, 16)]
                    m = (d >= lo) & (d < hi)
                    mi = m.astype(jnp.int32)
                    csum = plsc.cumsum(mi)
                    pos = off2 + csum - mi
                    eid = i * SCAN + j * 16 + iota16
                    plsc.store_scatter(ids, [pos], eid, mask=m)
                    plsc.store_scatter(rows, [pos], d - lo, mask=m)
                    off2 = off2 + plsc.all_reduce_population_count(m)[0]
                return off2
            off = lax.fori_loop(0, SCAN // 64, vreg4, off)
            off = lax.cond(off >= HIW, drain, lambda c: c, off)
            return off

        off = lax.fori_loop(0, n_scan, scan_chunk, jnp.int32(0))
        lax.cond(off > 0, drain, lambda c: c, off)

        pltpu.sync_copy(asum, osum.at[pl.ds(lo, NPER)])
        pltpu.sync_copy(amax, omax.at[pl.ds(lo, NPER)])
        pltpu.sync_copy(adeg.at[pl.ds(0, NPER)], odeg.at[pl.ds(lo, NPER)])

    return k(dst, ms)


CSUB = 40
CFIRE = 5
CCH = CSUB * CFIRE

def _vector_scatter(v0, v1, v2, zeros_n, src, mv0, mv1, mv2):
    per_t = E // NS
    n_ch1 = per_t // CCH
    n_ch2 = (E // 2) // NS // CCH
    RP = 624

    @functools.partial(
        pl.kernel, mesh=_mesh(), compiler_params=_sc_params,
        out_type=(
            jax.ShapeDtypeStruct((N, F), jnp.float32),
            jax.ShapeDtypeStruct((N, F), jnp.float32),
            jax.ShapeDtypeStruct((N, F), jnp.float32),
            jax.ShapeDtypeStruct((N, F), jnp.float32),
        ),
        scratch_types=[
            [pltpu.VMEM((CSUB,), jnp.int32) for _ in range(CFIRE)],
            pltpu.VMEM((CCH, F), jnp.float32),
            pltpu.VMEM_SHARED((N, F), jnp.float32),
            pltpu.SemaphoreType.DMA,
        ],
    )
    def k(v0_hbm, v1_hbm, v2_hbm, z_hbm, src_hbm, mv0_hbm, mv1_hbm, mv2_hbm,
          o0, o1, oa, ob, idx_bufs, upd_v, acc_sh, sem):
        c = lax.axis_index("c")
        s = lax.axis_index("s")
        rlo = s * RP
        last = N - (NS - 1) * RP

        def rowcopy(src_ref, dst_ref):
            @pl.when(s < NS - 1)
            def _():
                pltpu.sync_copy(src_ref.at[pl.ds(rlo, RP)],
                                dst_ref.at[pl.ds(rlo, RP)])

            @pl.when(s == NS - 1)
            def _():
                pltpu.sync_copy(src_ref.at[pl.ds(rlo, last)],
                                dst_ref.at[pl.ds(rlo, last)])

        def scatter_pass(mv_hbm, base, n_ch):
            def body(i, carry):
                off = base + i * CCH
                for g in range(CFIRE):
                    pltpu.sync_copy(src_hbm.at[pl.ds(off + g * CSUB, CSUB)],
                                    idx_bufs[g])
                pltpu.sync_copy(mv_hbm.at[pl.ds(off, CCH)], upd_v)
                cps = [
                    pltpu.async_copy(
                        upd_v.at[pl.ds(g * CSUB, CSUB)],
                        acc_sh.at[idx_bufs[g]], sem, add=True)
                    for g in range(CFIRE)
                ]
                for cp in cps:
                    cp.wait()
                return carry
            lax.fori_loop(0, n_ch, body, 0)

        @pl.when(c == 0)
        def _():
            rowcopy(v0_hbm, acc_sh)
        @pl.when(c == 1)
        def _():
            rowcopy(v1_hbm, acc_sh)
        plsc.subcore_barrier()
        @pl.when(c == 0)
        def _():
            scatter_pass(mv0_hbm, s * per_t, n_ch1)
        @pl.when(c == 1)
        def _():
            scatter_pass(mv1_hbm, s * per_t, n_ch1)
        plsc.subcore_barrier()
        @pl.when(c == 0)
        def _():
            rowcopy(acc_sh, o0)
        @pl.when(c == 1)
        def _():
            rowcopy(acc_sh, o1)
        plsc.subcore_barrier()

        @pl.when(c == 0)
        def _():
            rowcopy(v2_hbm, acc_sh)
        @pl.when(c == 1)
        def _():
            rowcopy(z_hbm, acc_sh)
        plsc.subcore_barrier()
        half_t = (E // 2) // NS
        scatter_pass(mv2_hbm, c * (E // 2) + s * half_t, n_ch2)
        plsc.subcore_barrier()
        @pl.when(c == 0)
        def _():
            rowcopy(acc_sh, oa)
        @pl.when(c == 1)
        def _():
            rowcopy(acc_sh, ob)

    return k(v0, v1, v2, zeros_n, src, mv0, mv1, mv2)


def _combine(a, b):
    CB = 200

    def body(a_r, b_r, o_r):
        o_r[...] = a_r[...] + b_r[...]

    return pl.pallas_call(
        body,
        grid=(N // CB,),
        in_specs=[pl.BlockSpec((CB, F), lambda i: (i, 0))] * 2,
        out_specs=pl.BlockSpec((CB, F), lambda i: (i, 0)),
        out_shape=jax.ShapeDtypeStruct((N, F), jnp.float32),
    )(a, b)


NB = 256

def _post(xpad, agg_sum, agg_max, deg_col, deg_row, W_post, b_post):
    nblk = NPAD // NB

    def body(x_r, s_r, m_r, dc_r, dr_r, wp_r, bp_r, out_r):
        d_all = dr_r[...]
        avg = jnp.sum(jnp.log1p(d_all)) / float(N)
        d = dc_r[...]
        x_blk = x_r[...]
        mean = s_r[...] / jnp.maximum(d, 1.0)
        maxz = jnp.where(d > 0.0, m_r[...], 0.0)
        r = jnp.log1p(d) / avg
        wp = wp_r[...]
        delta = (jnp.dot(x_blk, wp[0:F], preferred_element_type=jnp.float32)
                 + jnp.dot(mean, wp[F:2 * F], preferred_element_type=jnp.float32)
                 + jnp.dot(maxz, wp[2 * F:3 * F],
                           preferred_element_type=jnp.float32)
                 + jnp.dot(mean * r, wp[3 * F:4 * F],
                           preferred_element_type=jnp.float32)
                 + jnp.dot(maxz * r, wp[4 * F:5 * F],
                           preferred_element_type=jnp.float32)
                 + bp_r[...])
        out_r[...] = x_blk + delta

    return pl.pallas_call(
        body,
        grid=(nblk,),
        in_specs=[
            pl.BlockSpec((NB, F), lambda i: (i, 0)),
            pl.BlockSpec((NB, F), lambda i: (i, 0)),
            pl.BlockSpec((NB, F), lambda i: (i, 0)),
            pl.BlockSpec((NB, 1), lambda i: (i, 0)),
            pl.BlockSpec((1, NPAD), lambda i: (0, 0)),
            pl.BlockSpec((5 * F, F), lambda i: (0, 0)),
            pl.BlockSpec((1, F), lambda i: (0, 0)),
        ],
        out_specs=pl.BlockSpec((NB, F), lambda i: (i, 0)),
        out_shape=jax.ShapeDtypeStruct((NPAD, F), jnp.float32),
    )(xpad, agg_sum, agg_max, deg_col, deg_row, W_post,
      b_post.reshape(1, F))


def kernel(x, v, edge_index, edge_rbf, edge_vec, edge_attr,
           W_rbf_emb, b_rbf_emb, W_edge, b_edge, W_pre, b_pre,
           W_sm1, b_sm1, W_sm2, b_sm2, W_rbf_lin, W_post, b_post):
    src = edge_index[0]
    dst = edge_index[1]

    xsxd = _gather_rows(x, edge_index.reshape(2 * E))

    ms, mv0, mv1, mv2 = _edge_mlp(
        xsxd, edge_rbf, edge_attr, edge_vec,
        W_rbf_emb, b_rbf_emb, W_edge, b_edge, W_pre, b_pre,
        W_sm1, b_sm1, W_sm2, b_sm2, W_rbf_lin)

    agg_sum, agg_max, deg = _segment_agg(dst, ms)

    o0, o1, oa, ob = _vector_scatter(
        v[:, 0, :], v[:, 1, :], v[:, 2, :], jnp.zeros((N, F), jnp.float32),
        src, mv0, mv1, mv2)
    o2 = _combine(oa, ob)

    xpad = jnp.pad(x, ((0, NPAD - N), (0, 0)))
    xnew = _post(xpad, agg_sum, agg_max, deg.reshape(NPAD, 1),
                 deg.reshape(1, NPAD), W_post, b_post)[:N]

    vnew = jnp.stack([o0, o1, o2], axis=1)
    return (xnew, vnew)


# speedup vs baseline: 1.0598x; 1.0598x over previous
import functools

import jax
import jax.numpy as jnp
from jax import lax
from jax.experimental import pallas as pl
from jax.experimental.pallas import tpu as pltpu
from jax.experimental.pallas import tpu_sc as plsc

N = 10000
E = 320000
F = 128
NPAD = 10240

NC = 2
NS = 16
NW = NC * NS

_mesh = functools.partial(
    plsc.VectorSubcoreMesh, core_axis_name="c", subcore_axis_name="s")
_sc_params = pltpu.CompilerParams(needs_layout_passes=False)


GSUB = 80
GFIRE = 5
GCH = GSUB * GFIRE

def _gather_rows(table, idx):
    B = idx.shape[0]
    D = table.shape[1]
    per_w = B // NW
    n_ch = per_w // GCH

    @functools.partial(
        pl.kernel, mesh=_mesh(), compiler_params=_sc_params,
        out_type=jax.ShapeDtypeStruct((B, D), jnp.float32),
        scratch_types=[
            pltpu.VMEM((GCH,), jnp.int32),
            pltpu.VMEM((GCH, D), jnp.float32),
            pltpu.SemaphoreType.DMA,
        ],
    )
    def k(table_hbm, idx_hbm, out_hbm, idx_v, rows_v, sem):
        wid = lax.axis_index("s") * NC + lax.axis_index("c")
        base = wid * per_w

        def body(i, carry):
            off = base + i * GCH
            pltpu.sync_copy(idx_hbm.at[pl.ds(off, GCH)], idx_v)
            cps = [
                pltpu.async_copy(
                    table_hbm.at[idx_v.at[pl.ds(g * GSUB, GSUB)]],
                    rows_v.at[pl.ds(g * GSUB, GSUB)], sem)
                for g in range(GFIRE)
            ]
            for cp in cps:
                cp.wait()
            pltpu.sync_copy(rows_v, out_hbm.at[pl.ds(off, GCH)])
            return carry

        lax.fori_loop(0, n_ch, body, 0)

    return k(table, idx)


BE = 256

def _edge_mlp(xsxd, edge_rbf, edge_attr, edge_vec,
              W_rbf_emb, b_rbf_emb, W_edge, b_edge, W_pre, b_pre,
              W_sm1, b_sm1, W_sm2, b_sm2, W_rbf_lin):
    nblk = E // BE

    def body(xs_r, xd_r, rbf_r, ea_r, ev_r,
             wre_r, bre_r, we_r, be_r, wp_r, bp_r,
             w1_r, b1_r, w2_r, b2_r, wrl_r,
             ms_r, mv0_r, mv1_r, mv2_r):
        rbf = rbf_r[...]
        rbf_attr = jax.nn.relu(
            jnp.dot(rbf, wre_r[...], preferred_element_type=jnp.float32)
            + bre_r[...])
        ea = (jnp.dot(ea_r[...], we_r[...], preferred_element_type=jnp.float32)
              + be_r[...])
        wp = wp_r[...]
        xd = xd_r[...]
        msg = (jnp.dot(xs_r[...], wp[0:F], preferred_element_type=jnp.float32)
               + jnp.dot(xd, wp[F:2 * F], preferred_element_type=jnp.float32)
               + jnp.dot(rbf_attr, wp[2 * F:3 * F],
                         preferred_element_type=jnp.float32)
               + jnp.dot(ea, wp[3 * F:4 * F],
                         preferred_element_type=jnp.float32)
               + bp_r[...])
        h = jnp.dot(msg, w1_r[...], preferred_element_type=jnp.float32) + b1_r[...]
        h = h * jax.nn.sigmoid(h)
        so3 = jnp.dot(h, w2_r[...], preferred_element_type=jnp.float32) + b2_r[...]
        filt = so3 * jnp.dot(rbf, wrl_r[...], preferred_element_type=jnp.float32)
        gs = filt[:, 0:F]
        ge = filt[:, F:2 * F]
        ms_r[...] = filt[:, 2 * F:3 * F]
        ev = ev_r[...]
        a = xd * gs
        mv0_r[...] = a + ge * ev[:, 0:1]
        mv1_r[...] = a + ge * ev[:, 1:2]
        mv2_r[...] = a + ge * ev[:, 2:3]

    full = lambda r, c: pl.BlockSpec((r, c), lambda i: (0, 0))
    return pl.pallas_call(
        body,
        grid=(nblk,),
        in_specs=[
            pl.BlockSpec((BE, F), lambda i: (i, 0)),
            pl.BlockSpec((BE, F), lambda i: (i + nblk, 0)),
            pl.BlockSpec((BE, 20), lambda i: (i, 0)),
            pl.BlockSpec((BE, 16), lambda i: (i, 0)),
            pl.BlockSpec((BE, 3), lambda i: (i, 0)),
            full(20, F), full(1, F), full(16, F), full(1, F),
            full(4 * F, F), full(1, F),
            full(F, F), full(1, F), full(F, 3 * F), full(1, 3 * F),
            full(20, 3 * F),
        ],
        out_specs=[pl.BlockSpec((BE, F), lambda i: (i, 0))] * 4,
        out_shape=[jax.ShapeDtypeStruct((E, F), jnp.float32)] * 4,
    )(xsxd, xsxd, edge_rbf, edge_attr, edge_vec,
      W_rbf_emb, b_rbf_emb.reshape(1, F), W_edge, b_edge.reshape(1, F),
      W_pre, b_pre.reshape(1, F), W_sm1, b_sm1.reshape(1, F),
      W_sm2, b_sm2.reshape(1, 3 * F), W_rbf_lin)


NPER = 320
SCAN = 800
CAP = 4112
HIW = CAP - SCAN - 16
GB = 128
NEG = -3.0e38

def _segment_agg(dst, ms):
    n_scan = E // SCAN

    @functools.partial(
        pl.kernel, mesh=_mesh(), compiler_params=_sc_params,
        out_type=(
            jax.ShapeDtypeStruct((NPAD, F), jnp.float32),
            jax.ShapeDtypeStruct((NPAD, F), jnp.float32),
            jax.ShapeDtypeStruct((NPAD,), jnp.float32),
        ),
        scratch_types=[
            pltpu.VMEM((NPER, F), jnp.float32),
            pltpu.VMEM((NPER, F), jnp.float32),
            pltpu.VMEM((NPER + 16,), jnp.float32),
            pltpu.VMEM((SCAN,), jnp.int32),
            pltpu.VMEM((CAP,), jnp.int32),
            pltpu.VMEM((CAP,), jnp.int32),
            pltpu.VMEM((GB, F), jnp.float32),
            pltpu.SemaphoreType.DMA,
        ],
    )
    def k(dst_hbm, ms_hbm, osum, omax, odeg,
          asum, amax, adeg, scanb, ids, rows, gbuf, sem):
        wid = lax.axis_index("s") * NC + lax.axis_index("c")
        lo = wid * NPER
        hi = lo + NPER

        zero16 = jnp.zeros((16,), jnp.float32)
        neg16 = jnp.full((16,), NEG, jnp.float32)
        zi16 = jnp.zeros((16,), jnp.int32)

        def init_acc(i, carry):
            for q in range(F // 16):
                asum[i, pl.ds(q * 16, 16)] = zero16
                amax[i, pl.ds(q * 16, 16)] = neg16
            return carry
        lax.fori_loop(0, NPER, init_acc, 0)

        def init_deg(i, carry):
            adeg[pl.ds(i * 16, 16)] = zero16
            return carry
        lax.fori_loop(0, (NPER + 16) // 16, init_deg, 0)

        def init_ids(i, carry):
            ids[pl.ds(i * 16, 16)] = zi16
            return carry
        lax.fori_loop(0, CAP // 16, init_ids, 0)

        def drain(cnt):
            nsub = (cnt + GB - 1) // GB

            def sub(p, carry):
                base_r = p * GB
                pltpu.async_copy(
                    ms_hbm.at[ids.at[pl.ds(base_r, GB)]], gbuf, sem).wait()
                nr = jnp.minimum(GB, cnt - base_r)

                def rmw(r, c2):
                    row = rows[pl.ds(base_r + r, 16)][0]
                    for q in range(F // 16):
                        val = gbuf[r, pl.ds(q * 16, 16)]
                        am = amax[row, pl.ds(q * 16, 16)]
                        amax[row, pl.ds(q * 16, 16)] = jnp.maximum(am, val)
                        sm = asum[row, pl.ds(q * 16, 16)]
                        asum[row, pl.ds(q * 16, 16)] = sm + val
                    adeg[pl.ds(row, 16)] = adeg[pl.ds(row, 16)] + e0
                    return c2
                lax.fori_loop(0, nr, rmw, 0)
                return carry
            lax.fori_loop(0, nsub, sub, 0)
            return 0

        iota16 = lax.iota(jnp.int32, 16)
        e0 = (iota16 == 0).astype(jnp.float32)

        def scan_chunk(i, off):
            pltpu.sync_copy(dst_hbm.at[pl.ds(i * SCAN, SCAN)], scanb)

            def vreg(j, off2):
                d = scanb[pl.ds(j * 16, 16)]
                m = (d >= lo) & (d < hi)
                mi = m.astype(jnp.int32)
                csum = plsc.cumsum(mi)
                pos = off2 + csum - mi
                eid = i * SCAN + j * 16 + iota16
                plsc.store_scatter(ids, [pos], eid, mask=m)
                plsc.store_scatter(rows, [pos], d - lo, mask=m)
                return off2 + csum[15]
            off = lax.fori_loop(0, SCAN // 16, vreg, off)
            off = lax.cond(off >= HIW, drain, lambda c: c, off)
            return off

        off = lax.fori_loop(0, n_scan, scan_chunk, jnp.int32(0))
        lax.cond(off > 0, drain, lambda c: c, off)

        pltpu.sync_copy(asum, osum.at[pl.ds(lo, NPER)])
        pltpu.sync_copy(amax, omax.at[pl.ds(lo, NPER)])
        pltpu.sync_copy(adeg.at[pl.ds(0, NPER)], odeg.at[pl.ds(lo, NPER)])

    return k(dst, ms)


CSUB = 40
CFIRE = 5
CCH = CSUB * CFIRE

def _vector_scatter(v0, v1, v2, zeros_n, src, mv0, mv1, mv2):
    per_t = E // NS
    n_ch1 = per_t // CCH
    n_ch2 = (E // 2) // NS // CCH
    RP = 624

    @functools.partial(
        pl.kernel, mesh=_mesh(), compiler_params=_sc_params,
        out_type=(
            jax.ShapeDtypeStruct((N, F), jnp.float32),
            jax.ShapeDtypeStruct((N, F), jnp.float32),
            jax.ShapeDtypeStruct((N, F), jnp.float32),
            jax.ShapeDtypeStruct((N, F), jnp.float32),
        ),
        scratch_types=[
            [pltpu.VMEM((CSUB,), jnp.int32) for _ in range(CFIRE)],
            pltpu.VMEM((CCH, F), jnp.float32),
            pltpu.VMEM_SHARED((N, F), jnp.float32),
            pltpu.SemaphoreType.DMA,
        ],
    )
    def k(v0_hbm, v1_hbm, v2_hbm, z_hbm, src_hbm, mv0_hbm, mv1_hbm, mv2_hbm,
          o0, o1, oa, ob, idx_bufs, upd_v, acc_sh, sem):
        c = lax.axis_index("c")
        s = lax.axis_index("s")
        rlo = s * RP
        last = N - (NS - 1) * RP

        def rowcopy(src_ref, dst_ref):
            @pl.when(s < NS - 1)
            def _():
                pltpu.sync_copy(src_ref.at[pl.ds(rlo, RP)],
                                dst_ref.at[pl.ds(rlo, RP)])

            @pl.when(s == NS - 1)
            def _():
                pltpu.sync_copy(src_ref.at[pl.ds(rlo, last)],
                                dst_ref.at[pl.ds(rlo, last)])

        def scatter_pass(mv_hbm, base, n_ch):
            def body(i, carry):
                off = base + i * CCH
                for g in range(CFIRE):
                    pltpu.sync_copy(src_hbm.at[pl.ds(off + g * CSUB, CSUB)],
                                    idx_bufs[g])
                pltpu.sync_copy(mv_hbm.at[pl.ds(off, CCH)], upd_v)
                cps = [
                    pltpu.async_copy(
                        upd_v.at[pl.ds(g * CSUB, CSUB)],
                        acc_sh.at[idx_bufs[g]], sem, add=True)
                    for g in range(CFIRE)
                ]
                for cp in cps:
                    cp.wait()
                return carry
            lax.fori_loop(0, n_ch, body, 0)

        @pl.when(c == 0)
        def _():
            rowcopy(v0_hbm, acc_sh)
        @pl.when(c == 1)
        def _():
            rowcopy(v1_hbm, acc_sh)
        plsc.subcore_barrier()
        @pl.when(c == 0)
        def _():
            scatter_pass(mv0_hbm, s * per_t, n_ch1)
        @pl.when(c == 1)
        def _():
            scatter_pass(mv1_hbm, s * per_t, n_ch1)
        plsc.subcore_barrier()
        @pl.when(c == 0)
        def _():
            rowcopy(acc_sh, o0)
        @pl.when(c == 1)
        def _():
            rowcopy(acc_sh, o1)
        plsc.subcore_barrier()

        @pl.when(c == 0)
        def _():
            rowcopy(v2_hbm, acc_sh)
        @pl.when(c == 1)
        def _():
            rowcopy(z_hbm, acc_sh)
        plsc.subcore_barrier()
        half_t = (E // 2) // NS
        scatter_pass(mv2_hbm, c * (E // 2) + s * half_t, n_ch2)
        plsc.subcore_barrier()
        @pl.when(c == 0)
        def _():
            rowcopy(acc_sh, oa)
        @pl.when(c == 1)
        def _():
            rowcopy(acc_sh, ob)

    return k(v0, v1, v2, zeros_n, src, mv0, mv1, mv2)


def _combine(a, b):
    CB = 200

    def body(a_r, b_r, o_r):
        o_r[...] = a_r[...] + b_r[...]

    return pl.pallas_call(
        body,
        grid=(N // CB,),
        in_specs=[pl.BlockSpec((CB, F), lambda i: (i, 0))] * 2,
        out_specs=pl.BlockSpec((CB, F), lambda i: (i, 0)),
        out_shape=jax.ShapeDtypeStruct((N, F), jnp.float32),
    )(a, b)


NB = 256

def _post(xpad, agg_sum, agg_max, deg_col, deg_row, W_post, b_post):
    nblk = NPAD // NB

    def body(x_r, s_r, m_r, dc_r, dr_r, wp_r, bp_r, out_r):
        d_all = dr_r[...]
        avg = jnp.sum(jnp.log1p(d_all)) / float(N)
        d = dc_r[...]
        x_blk = x_r[...]
        mean = s_r[...] / jnp.maximum(d, 1.0)
        maxz = jnp.where(d > 0.0, m_r[...], 0.0)
        r = jnp.log1p(d) / avg
        wp = wp_r[...]
        delta = (jnp.dot(x_blk, wp[0:F], preferred_element_type=jnp.float32)
                 + jnp.dot(mean, wp[F:2 * F], preferred_element_type=jnp.float32)
                 + jnp.dot(maxz, wp[2 * F:3 * F],
                           preferred_element_type=jnp.float32)
                 + jnp.dot(mean * r, wp[3 * F:4 * F],
                           preferred_element_type=jnp.float32)
                 + jnp.dot(maxz * r, wp[4 * F:5 * F],
                           preferred_element_type=jnp.float32)
                 + bp_r[...])
        out_r[...] = x_blk + delta

    return pl.pallas_call(
        body,
        grid=(nblk,),
        in_specs=[
            pl.BlockSpec((NB, F), lambda i: (i, 0)),
            pl.BlockSpec((NB, F), lambda i: (i, 0)),
            pl.BlockSpec((NB, F), lambda i: (i, 0)),
            pl.BlockSpec((NB, 1), lambda i: (i, 0)),
            pl.BlockSpec((1, NPAD), lambda i: (0, 0)),
            pl.BlockSpec((5 * F, F), lambda i: (0, 0)),
            pl.BlockSpec((1, F), lambda i: (0, 0)),
        ],
        out_specs=pl.BlockSpec((NB, F), lambda i: (i, 0)),
        out_shape=jax.ShapeDtypeStruct((NPAD, F), jnp.float32),
    )(xpad, agg_sum, agg_max, deg_col, deg_row, W_post,
      b_post.reshape(1, F))


def kernel(x, v, edge_index, edge_rbf, edge_vec, edge_attr,
           W_rbf_emb, b_rbf_emb, W_edge, b_edge, W_pre, b_pre,
           W_sm1, b_sm1, W_sm2, b_sm2, W_rbf_lin, W_post, b_post):
    src = edge_index[0]
    dst = edge_index[1]

    xsxd = _gather_rows(x, edge_index.reshape(2 * E))

    ms, mv0, mv1, mv2 = _edge_mlp(
        xsxd, edge_rbf, edge_attr, edge_vec,
        W_rbf_emb, b_rbf_emb, W_edge, b_edge, W_pre, b_pre,
        W_sm1, b_sm1, W_sm2, b_sm2, W_rbf_lin)

    agg_sum, agg_max, deg = _segment_agg(dst, ms)

    o0, o1, oa, ob = _vector_scatter(
        v[:, 0, :], v[:, 1, :], v[:, 2, :], jnp.zeros((N, F), jnp.float32),
        src, mv0, mv1, mv2)
    o2 = _combine(oa, ob)

    xpad = jnp.pad(x, ((0, NPAD - N), (0, 0)))
    xnew = _post(xpad, agg_sum, agg_max, deg.reshape(NPAD, 1),
                 deg.reshape(1, NPAD), W_post, b_post)[:N]

    vnew = jnp.stack([o0, o1, o2], axis=1)
    return (xnew, vnew)
